# baseline (speedup 1.0000x reference)
import functools

import jax
import jax.numpy as jnp
from jax import lax
from jax.experimental import pallas as pl
from jax.experimental.pallas import tpu as pltpu
from jax.experimental.pallas import tpu_sc as plsc

K_RATIO = 0.04
SIM_THRESHOLD = 0.6
KP = 176


def _score_body(x_ref, d_ref, *, num_select, hw, w):
    nr = hw // w
    xb = x_ref[0]
    s = jnp.mean(jnp.abs(xb), axis=0)
    s_bits = lax.bitcast_convert_type(s, jnp.int32)

    def vstep(i, t):
        cand = t | (1 << (30 - i))
        c = jnp.sum((s_bits >= cand).astype(jnp.int32))
        return jnp.where(c >= num_select, cand, t)

    v = lax.fori_loop(0, 31, vstep, jnp.int32(0))
    c_gt = jnp.sum((s_bits > v).astype(jnp.int32))
    rem = num_select - c_gt
    eq = s_bits == v
    rny = lax.broadcasted_iota(jnp.int32, (nr, w), 0)
    rnx = lax.broadcasted_iota(jnp.int32, (nr, w), 1)
    pix = rny * w + rnx

    def pstep(i, lohi):
        lo, hi = lohi
        mid = (lo + hi) // 2
        c = jnp.sum((eq & (pix <= mid)).astype(jnp.int32))
        return jnp.where(c >= rem, lo, mid + 1), jnp.where(c >= rem, mid, hi)

    _, pv = lax.fori_loop(0, 12, pstep, (jnp.int32(0), jnp.int32(hw - 1)))

    sel2d = ((s_bits > v) | (eq & (pix <= pv))).astype(jnp.float32)
    rw = lax.broadcasted_iota(jnp.int32, (w, w), 0)
    cw = lax.broadcasted_iota(jnp.int32, (w, w), 1)
    ut = (rw <= cw).astype(jnp.float32)
    incl = jnp.dot(sel2d, ut, preferred_element_type=jnp.float32)
    rnr = lax.broadcasted_iota(jnp.int32, (nr, nr), 0)
    cnr = lax.broadcasted_iota(jnp.int32, (nr, nr), 1)
    lt = (cnr < rnr).astype(jnp.float32)
    plast = (rw == (w - 1)).astype(jnp.float32)
    tot = jnp.dot(incl, plast, preferred_element_type=jnp.float32)
    base = jnp.dot(lt, tot, preferred_element_type=jnp.float32)
    rank = incl - sel2d + base
    d_ref[0] = jnp.where(sel2d > 0.0, rank, 16384.0).astype(jnp.int32)


def _dense_body(f_ref, w_ref, b_ref, u_ref):
    F = f_ref[...]
    norm = jnp.sqrt(jnp.sum(F * F, axis=0, keepdims=True))
    nrm = F / (norm + 1e-12)
    sim = lax.dot_general(nrm, nrm, (((0,), (0,)), ((), ())),
                          preferred_element_type=jnp.float32)
    inv = (1.0 - sim) * 0.5
    thresh = (1.0 - SIM_THRESHOLD) / 2.0
    A = jnp.where(inv < thresh, inv, 0.0)
    hT = lax.dot_general(w_ref[...], F, (((0,), (0,)), ((), ())),
                         preferred_element_type=jnp.float32) + b_ref[...]
    u_ref[...] = jnp.maximum(
        lax.dot_general(hT, A, (((1,), (0,)), ((), ())),
                        preferred_element_type=jnp.float32), 0.0)


def _wid_map():
    cid = lax.axis_index("c")
    sid = lax.axis_index("s")
    wid = sid * 2 + cid
    return wid // 4, (wid % 4) * 48


def _al(off):
    return pl.multiple_of(off, 8)


def _compact_indices(drow, idxb, lane, HW):
    def zstep(j, _):
        idxb[pl.ds(j * 16, 16)] = jnp.zeros((16,), jnp.int32)
        return 0

    lax.fori_loop(0, 12, zstep, 0)

    def cstep(j, _):
        d = drow[pl.ds(j * 16, 16)]
        pix = lane + j * 16
        plsc.store_scatter(idxb, [d], pix, mask=d < jnp.int32(16384))
        return 0

    lax.fori_loop(0, HW // 16, cstep, 0)


def _gather_body(dest_hbm, xf_hbm, f_hbm, idx_hbm,
                 drow, idxb, elist, fbuf, gsem, wsem, *, C, HW, num_select):
    b, c0 = _wid_map()
    lane = lax.iota(jnp.int32, 16)
    pltpu.sync_copy(dest_hbm.at[pl.ds(_al(b * HW), HW)], drow)
    _compact_indices(drow, idxb, lane, HW)

    @pl.when(c0 == 0)
    def _():
        pltpu.sync_copy(idxb.at[pl.ds(0, KP)],
                        idx_hbm.at[pl.ds(_al(b * KP), KP)])

    def build(c, _):
        base = (b * C + c0 + c) * HW

        def jstep(j, _):
            idxv = idxb[pl.ds(j * 16, 16)]
            elist[pl.ds(c * KP + j * 16, 16)] = idxv + base
            return 0

        lax.fori_loop(0, KP // 16, jstep, 0)
        return 0

    lax.fori_loop(0, 48, build, 0)

    ng = 48 * KP // 128

    def fire(g, _):
        pltpu.make_async_copy(xf_hbm.at[elist.at[pl.ds(g * 128, 128)]],
                              fbuf.at[pl.ds(g * 128, 128)], gsem).start()
        return 0

    lax.fori_loop(0, ng, fire, 0)

    def drain(g, _):
        pltpu.make_async_copy(xf_hbm.at[elist.at[pl.ds(g * 128, 128)]],
                              fbuf.at[pl.ds(g * 128, 128)], gsem).wait()
        return 0

    lax.fori_loop(0, ng, drain, 0)

    def wstep(c, _):
        pos = lane + 160
        v = fbuf[pl.ds(c * KP + 160, 16)]
        fbuf[pl.ds(c * KP + 160, 16)] = jnp.where(pos < num_select, v, 0.0)
        pltpu.make_async_copy(
            fbuf.at[pl.ds(c * KP, KP)],
            f_hbm.at[pl.ds(_al((c0 + c) * (8 * KP) + b * KP), KP)],
            wsem).start()
        return 0

    lax.fori_loop(0, 48, wstep, 0)

    def wdrain(c, _):
        pltpu.make_async_copy(
            fbuf.at[pl.ds(c * KP, KP)],
            f_hbm.at[pl.ds(_al((c0 + c) * (8 * KP) + b * KP), KP)],
            wsem).wait()
        return 0

    lax.fori_loop(0, 48, wdrain, 0)


def _scatter_body(xf_hbm, idx_hbm, u_hbm, out_hbm,
                  idxb, ub, rowb, rsem, wsem, *, C, HW, num_select):
    b, c0 = _wid_map()
    lane = lax.iota(jnp.int32, 16)
    pltpu.sync_copy(idx_hbm.at[pl.ds(_al(b * KP), KP)], idxb)
    pltpu.sync_copy(u_hbm.at[pl.ds(_al((b * C + c0) * KP), 48 * KP)], ub)

    def rd(c, k):
        return pltpu.make_async_copy(
            xf_hbm.at[pl.ds(_al((b * C + c0 + c) * HW), HW)],
            rowb.at[pl.ds(k * HW, HW)], rsem.at[k])

    def wr(c, k):
        return pltpu.make_async_copy(
            rowb.at[pl.ds(k * HW, HW)],
            out_hbm.at[pl.ds(_al((b * C + c0 + c) * HW), HW)],
            wsem.at[k])

    rd(0, 0).start()
    rd(1, 1).start()

    def rowstep(c, _):
        k = c % 2
        rd(c, k).wait()

        def jstep(j, _):
            pos = lane + j * 16
            idxv = idxb[pl.ds(j * 16, 16)]
            vals = ub[pl.ds(c * KP + j * 16, 16)]
            plsc.store_scatter(rowb.at[pl.ds(k * HW, HW)], [idxv], vals,
                               mask=pos < num_select)
            return 0

        lax.fori_loop(0, KP // 16, jstep, 0)
        wr(c, k).start()

        @pl.when(c + 2 < 48)
        def _():
            wr(c, k).wait()
            rd(c + 2, k).start()

        return 0

    lax.fori_loop(0, 48, rowstep, 0)
    wr(46, 0).wait()
    wr(47, 1).wait()


def kernel(x, W_gcn, b_gcn):
    B, C, H, W = x.shape
    HW = H * W
    num_select = int(HW * K_RATIO)
    xf2 = x.reshape(B * C, HW)

    dest3 = pl.pallas_call(
        functools.partial(_score_body, num_select=num_select, hw=HW, w=W),
        grid=(B,),
        in_specs=[pl.BlockSpec((1, C, H, W), lambda b: (b, 0, 0, 0))],
        out_specs=pl.BlockSpec((1, H, W), lambda b: (b, 0, 0)),
        out_shape=jax.ShapeDtypeStruct((B, H, W), jnp.int32),
    )(x)

    mesh = plsc.VectorSubcoreMesh(core_axis_name="c", subcore_axis_name="s")
    F_flat, idxp = pl.kernel(
        functools.partial(_gather_body, C=C, HW=HW, num_select=num_select),
        out_type=[jax.ShapeDtypeStruct((C * B * KP,), jnp.float32),
                  jax.ShapeDtypeStruct((B * KP,), jnp.int32)],
        mesh=mesh,
        compiler_params=pltpu.CompilerParams(needs_layout_passes=False),
        scratch_types=[pltpu.VMEM((HW,), jnp.int32),
                       pltpu.VMEM((192,), jnp.int32),
                       pltpu.VMEM((48 * KP,), jnp.int32),
                       pltpu.VMEM((48 * KP,), jnp.float32),
                       pltpu.SemaphoreType.DMA,
                       pltpu.SemaphoreType.DMA],
    )(dest3.reshape(-1), xf2.reshape(-1))
    F_cm = F_flat.reshape(C, B * KP)

    U_cm = pl.pallas_call(
        _dense_body,
        out_shape=jax.ShapeDtypeStruct((C, B * KP), jnp.float32),
    )(F_cm, W_gcn, b_gcn.reshape(C, 1))
    U_t = U_cm.reshape(C, B, KP).transpose(1, 0, 2).reshape(B * C * KP)

    out2 = pl.kernel(
        functools.partial(_scatter_body, C=C, HW=HW, num_select=num_select),
        out_type=jax.ShapeDtypeStruct((B * C * HW,), jnp.float32),
        mesh=mesh,
        compiler_params=pltpu.CompilerParams(needs_layout_passes=False),
        scratch_types=[pltpu.VMEM((KP,), jnp.int32),
                       pltpu.VMEM((48 * KP,), jnp.float32),
                       pltpu.VMEM((2 * HW,), jnp.float32),
                       pltpu.SemaphoreType.DMA((2,)),
                       pltpu.SemaphoreType.DMA((2,))],
    )(xf2.reshape(-1), idxp, U_t)
    return out2.reshape(B, C, H, W)


# speedup vs baseline: 1.0436x; 1.0436x over previous
import functools

import jax
import jax.numpy as jnp
from jax import lax
from jax.experimental import pallas as pl
from jax.experimental.pallas import tpu as pltpu
from jax.experimental.pallas import tpu_sc as plsc

K_RATIO = 0.04
SIM_THRESHOLD = 0.6
KP = 176


def _score_body(x_ref, d_ref, *, num_select, hw):
    nr = hw // 128
    xb = x_ref[0]
    s = jnp.mean(jnp.abs(xb), axis=0)
    s_bits = lax.bitcast_convert_type(s, jnp.int32)

    def vstep(i, t):
        cand = t | (1 << (30 - i))
        c = jnp.sum((s_bits >= cand).astype(jnp.int32))
        return jnp.where(c >= num_select, cand, t)

    v = lax.fori_loop(0, 31, vstep, jnp.int32(0))
    c_gt = jnp.sum((s_bits > v).astype(jnp.int32))
    rem = num_select - c_gt
    eq = s_bits == v
    rnr128 = lax.broadcasted_iota(jnp.int32, (nr, 128), 0)
    cnr128 = lax.broadcasted_iota(jnp.int32, (nr, 128), 1)
    pix = rnr128 * 128 + cnr128

    def pstep(i, lohi):
        lo, hi = lohi
        mid = (lo + hi) // 2
        c = jnp.sum((eq & (pix <= mid)).astype(jnp.int32))
        return jnp.where(c >= rem, lo, mid + 1), jnp.where(c >= rem, mid, hi)

    _, pv = lax.fori_loop(0, 12, pstep, (jnp.int32(0), jnp.int32(hw - 1)))

    sel2d = ((s_bits > v) | (eq & (pix <= pv))).astype(jnp.float32)
    r128 = lax.broadcasted_iota(jnp.int32, (128, 128), 0)
    c128 = lax.broadcasted_iota(jnp.int32, (128, 128), 1)
    ut = (r128 <= c128).astype(jnp.float32)
    incl = jnp.dot(sel2d, ut, preferred_element_type=jnp.float32)
    rnr = lax.broadcasted_iota(jnp.int32, (nr, nr), 0)
    cnr = lax.broadcasted_iota(jnp.int32, (nr, nr), 1)
    lt = (cnr < rnr).astype(jnp.float32)
    p127 = (r128 == 127).astype(jnp.float32)
    tot = jnp.dot(incl, p127, preferred_element_type=jnp.float32)
    base = jnp.dot(lt, tot, preferred_element_type=jnp.float32)
    rank = incl - sel2d + base
    d_ref[0] = jnp.where(sel2d > 0.0, rank, 16384.0).astype(jnp.int32)


def _dense_body(f_ref, w_ref, b_ref, u_ref):
    F = f_ref[...]
    norm = jnp.sqrt(jnp.sum(F * F, axis=0, keepdims=True))
    nrm = F / (norm + 1e-12)
    sim = lax.dot_general(nrm, nrm, (((0,), (0,)), ((), ())),
                          preferred_element_type=jnp.float32)
    inv = (1.0 - sim) * 0.5
    thresh = (1.0 - SIM_THRESHOLD) / 2.0
    A = jnp.where(inv < thresh, inv, 0.0)
    hT = lax.dot_general(w_ref[...], F, (((0,), (0,)), ((), ())),
                         preferred_element_type=jnp.float32) + b_ref[...]
    u_ref[...] = jnp.maximum(
        lax.dot_general(hT, A, (((1,), (0,)), ((), ())),
                        preferred_element_type=jnp.float32), 0.0)


def _wid_map():
    cid = lax.axis_index("c")
    sid = lax.axis_index("s")
    wid = sid * 2 + cid
    return wid // 4, (wid % 4) * 48


def _al(off):
    return pl.multiple_of(off, 8)


def _compact_indices(drow, idxb, lane, HW):
    def zstep(j, _):
        idxb[pl.ds(j * 16, 16)] = jnp.zeros((16,), jnp.int32)
        return 0

    lax.fori_loop(0, 12, zstep, 0)

    def cstep(j, _):
        d = drow[pl.ds(j * 16, 16)]
        pix = lane + j * 16
        plsc.store_scatter(idxb, [d], pix, mask=d < jnp.int32(16384))
        return 0

    lax.fori_loop(0, HW // 16, cstep, 0)


def _gather_body(dest_hbm, xf_hbm, f_hbm, idx_hbm,
                 drow, idxb, elist, fbuf, gsem, wsem, *, C, HW, num_select):
    b, c0 = _wid_map()
    lane = lax.iota(jnp.int32, 16)
    pltpu.sync_copy(dest_hbm.at[pl.ds(_al(b * HW), HW)], drow)
    _compact_indices(drow, idxb, lane, HW)

    @pl.when(c0 == 0)
    def _():
        pltpu.sync_copy(idxb.at[pl.ds(0, KP)],
                        idx_hbm.at[pl.ds(_al(b * KP), KP)])

    def build(c, _):
        base = (b * C + c0 + c) * HW

        def jstep(j, _):
            idxv = idxb[pl.ds(j * 16, 16)]
            elist[pl.ds(c * KP + j * 16, 16)] = idxv + base
            return 0

        lax.fori_loop(0, KP // 16, jstep, 0)
        return 0

    lax.fori_loop(0, 48, build, 0)

    ng = 48 * KP // 128

    def fire(g, _):
        pltpu.make_async_copy(xf_hbm.at[elist.at[pl.ds(g * 128, 128)]],
                              fbuf.at[pl.ds(g * 128, 128)], gsem).start()
        return 0

    lax.fori_loop(0, ng, fire, 0)

    def drain(g, _):
        pltpu.make_async_copy(xf_hbm.at[elist.at[pl.ds(g * 128, 128)]],
                              fbuf.at[pl.ds(g * 128, 128)], gsem).wait()
        return 0

    lax.fori_loop(0, ng, drain, 0)

    def wstep(c, _):
        pos = lane + 160
        v = fbuf[pl.ds(c * KP + 160, 16)]
        fbuf[pl.ds(c * KP + 160, 16)] = jnp.where(pos < num_select, v, 0.0)
        pltpu.make_async_copy(
            fbuf.at[pl.ds(c * KP, KP)],
            f_hbm.at[pl.ds(_al((c0 + c) * (8 * KP) + b * KP), KP)],
            wsem).start()
        return 0

    lax.fori_loop(0, 48, wstep, 0)

    def wdrain(c, _):
        pltpu.make_async_copy(
            fbuf.at[pl.ds(c * KP, KP)],
            f_hbm.at[pl.ds(_al((c0 + c) * (8 * KP) + b * KP), KP)],
            wsem).wait()
        return 0

    lax.fori_loop(0, 48, wdrain, 0)


def _scatter_body(xf_hbm, idx_hbm, u_hbm, out_hbm,
                  idxb, ub, rowb, rsem, wsem, *, C, HW, num_select):
    b, c0 = _wid_map()
    lane = lax.iota(jnp.int32, 16)
    pltpu.sync_copy(idx_hbm.at[pl.ds(_al(b * KP), KP)], idxb)
    pltpu.sync_copy(u_hbm.at[pl.ds(_al((b * C + c0) * KP), 48 * KP)], ub)

    def rd(c, k):
        return pltpu.make_async_copy(
            xf_hbm.at[pl.ds(_al((b * C + c0 + c) * HW), HW)],
            rowb.at[pl.ds(k * HW, HW)], rsem.at[k])

    def wr(c, k):
        return pltpu.make_async_copy(
            rowb.at[pl.ds(k * HW, HW)],
            out_hbm.at[pl.ds(_al((b * C + c0 + c) * HW), HW)],
            wsem.at[k])

    rd(0, 0).start()
    rd(1, 1).start()

    def rowstep(c, _):
        k = c % 3
        rd(c, k).wait()

        def jstep(j, _):
            pos = lane + j * 16
            idxv = idxb[pl.ds(j * 16, 16)]
            vals = ub[pl.ds(c * KP + j * 16, 16)]
            plsc.store_scatter(rowb.at[pl.ds(k * HW, HW)], [idxv], vals,
                               mask=pos < num_select)
            return 0

        lax.fori_loop(0, KP // 16, jstep, 0)
        wr(c, k).start()

        @pl.when(c + 2 < 48)
        def _():
            kk = (c + 2) % 3

            @pl.when(c >= 1)
            def _():
                wr(c - 1, kk).wait()

            rd(c + 2, kk).start()

        return 0

    lax.fori_loop(0, 48, rowstep, 0)
    wr(45, 0).wait()
    wr(46, 1).wait()
    wr(47, 2).wait()


def kernel(x, W_gcn, b_gcn):
    B, C, H, W = x.shape
    HW = H * W
    num_select = int(HW * K_RATIO)
    xf2 = x.reshape(B * C, HW)

    nr = HW // 128
    x4 = x.reshape(B, C, nr, 128)
    dest3 = pl.pallas_call(
        functools.partial(_score_body, num_select=num_select, hw=HW),
        grid=(B,),
        in_specs=[pl.BlockSpec((1, C, nr, 128), lambda b: (b, 0, 0, 0))],
        out_specs=pl.BlockSpec((1, nr, 128), lambda b: (b, 0, 0)),
        out_shape=jax.ShapeDtypeStruct((B, nr, 128), jnp.int32),
    )(x4)

    mesh = plsc.VectorSubcoreMesh(core_axis_name="c", subcore_axis_name="s")
    F_flat, idxp = pl.kernel(
        functools.partial(_gather_body, C=C, HW=HW, num_select=num_select),
        out_type=[jax.ShapeDtypeStruct((C * B * KP,), jnp.float32),
                  jax.ShapeDtypeStruct((B * KP,), jnp.int32)],
        mesh=mesh,
        compiler_params=pltpu.CompilerParams(needs_layout_passes=False),
        scratch_types=[pltpu.VMEM((HW,), jnp.int32),
                       pltpu.VMEM((192,), jnp.int32),
                       pltpu.VMEM((48 * KP,), jnp.int32),
                       pltpu.VMEM((48 * KP,), jnp.float32),
                       pltpu.SemaphoreType.DMA,
                       pltpu.SemaphoreType.DMA],
    )(dest3.reshape(-1), xf2.reshape(-1))
    F_cm = F_flat.reshape(C, B * KP)

    U_cm = pl.pallas_call(
        _dense_body,
        out_shape=jax.ShapeDtypeStruct((C, B * KP), jnp.float32),
    )(F_cm, W_gcn, b_gcn.reshape(C, 1))
    U_t = U_cm.reshape(C, B, KP).transpose(1, 0, 2).reshape(B * C * KP)

    out2 = pl.kernel(
        functools.partial(_scatter_body, C=C, HW=HW, num_select=num_select),
        out_type=jax.ShapeDtypeStruct((B * C * HW,), jnp.float32),
        mesh=mesh,
        compiler_params=pltpu.CompilerParams(needs_layout_passes=False),
        scratch_types=[pltpu.VMEM((KP,), jnp.int32),
                       pltpu.VMEM((48 * KP,), jnp.float32),
                       pltpu.VMEM((3 * HW,), jnp.float32),
                       pltpu.SemaphoreType.DMA((3,)),
                       pltpu.SemaphoreType.DMA((3,))],
    )(xf2.reshape(-1), idxp, U_t)
    return out2.reshape(B, C, H, W)
